# baseline (speedup 1.0000x reference)
import functools

import jax
import jax.numpy as jnp
from jax import lax
from jax.experimental import pallas as pl
from jax.experimental.pallas import tpu as pltpu
from jax.experimental.pallas import tpu_sc as plsc

_N = [16384, 4096, 1024, 256, 64]
_ENC_F = [3, 16, 16, 16, 32]
_DEC = [(32, 16), (16, 16), (16, 16), (16, 16), (16, 3)]
_K = 6
_Z = 64
_B = 16
_KB = 128

_f32 = jnp.float32


_NB = 4
_WIN = 32


@functools.lru_cache(None)
def _make_spmv(n, R, E, C):
    mesh = plsc.VectorSubcoreMesh(core_axis_name="c", subcore_axis_name="s")
    n16 = n // 16
    Cout = 2 if C == 1 else C
    ept = E // 32 if C == 1 else E // 16
    KB = 128 if R <= 64 else (64 if R <= 256 else 32)
    KB = min(KB, ept)
    nblk = ept // KB
    assert ept % KB == 0 and n % 16 == 0

    def body(x_ref, edge_ref, z_ref, out_ref, *scr):
        eidx = scr[0:_NB]
        rows = scr[_NB:2 * _NB]
        isems = scr[2 * _NB:3 * _NB]
        gsems = scr[3 * _NB:4 * _NB]
        ssems = scr[4 * _NB:5 * _NB]
        acc = scr[5 * _NB]
        c = lax.axis_index("c")
        s = lax.axis_index("s")

        def pipe_window(xoff, base, W):
            idesc, gdesc, sdesc = {}, {}, {}
            for j in range(W + 2):
                if j < W:
                    b = j % _NB
                    if j >= _NB:
                        sdesc[j - _NB].wait()
                    idesc[j] = pltpu.async_copy(
                        edge_ref.at[:, pl.ds(base + j * KB, KB)],
                        eidx[b], isems[b])
                jg = j - 1
                if 0 <= jg < W:
                    bg = jg % _NB
                    idesc[jg].wait()
                    if C > 1:
                        for v in range(KB // 16):
                            sl = pl.ds(v * 16, 16)
                            eidx[bg][0, sl] = eidx[bg][0, sl] + xoff
                    gdesc[jg] = pltpu.async_copy(
                        x_ref.at[eidx[bg].at[0]], rows[bg], gsems[bg])
                js = j - 2
                if 0 <= js < W:
                    bs = js % _NB
                    gdesc[js].wait()
                    sdesc[js] = pltpu.async_copy(
                        rows[bs], acc.at[eidx[bs].at[1]], ssems[bs], add=True)
            for j in range(max(0, W - _NB), W):
                sdesc[j].wait()

        for jc in range(C // 2 if C > 1 else 1):
            p = (2 * jc + c) if C > 1 else 0
            pltpu.sync_copy(z_ref, acc.at[pl.ds(s * n16, n16), :])
            plsc.subcore_barrier()
            if C == 1:
                e_base = c * (E // 2) + s * ept
            else:
                e_base = s * ept
            xoff = p * n
            if nblk <= _WIN:
                pipe_window(xoff, e_base, nblk)
            else:
                def outer(w, carry, xoff=xoff, e_base=e_base):
                    pipe_window(xoff, e_base + w * (_WIN * KB), _WIN)
                    return carry
                lax.fori_loop(0, nblk // _WIN, outer, 0)
            plsc.subcore_barrier()
            obase = (c if C == 1 else p) * n
            pltpu.sync_copy(acc.at[pl.ds(s * n16, n16), :],
                            out_ref.at[pl.ds(obase + s * n16, n16), :])

    assert nblk <= _WIN or nblk % _WIN == 0
    scratch = ([pltpu.VMEM((2, KB), jnp.int32) for _ in range(_NB)]
               + [pltpu.VMEM((KB, R), _f32) for _ in range(_NB)]
               + [pltpu.SemaphoreType.DMA for _ in range(3 * _NB)]
               + [pltpu.VMEM_SHARED((n, R), _f32)])
    return pl.kernel(
        body,
        out_type=jax.ShapeDtypeStruct((Cout * n, R), _f32),
        mesh=mesh,
        compiler_params=pltpu.CompilerParams(use_tc_tiling_on_sc=False),
        scratch_types=scratch,
    )


def _spmv(x, edge, n, R, E, C=1):
    zeros = jnp.zeros((n // 16, R), _f32)
    return _make_spmv(n, R, E, C)(x, edge, zeros)


@functools.lru_cache(None)
def _make_gather(n_src, R, n_out, C, scale, coff):
    mesh = plsc.VectorSubcoreMesh(core_axis_name="c", subcore_axis_name="s")
    total = C * n_out
    m = max(total // 32, 8)
    active = total // m
    kb = min(_KB, m)
    nblk = m // kb
    tpc = active // C

    def body(x_ref, gidx_ref, out_ref, idxb, rows, sem):
        c = lax.axis_index("c")
        s = lax.axis_index("s")
        w = s * 2 + c

        def run():
            c_chunk = w // tpc
            i0 = (w % tpc) * m

            def blk(j, carry):
                r0 = i0 + j * kb
                pltpu.sync_copy(gidx_ref.at[pl.ds(r0, kb)], idxb)
                if scale != 1 or C > 1 or coff:
                    for v in range(kb // 16):
                        sl = pl.ds(v * 16, 16)
                        idxb[sl] = idxb[sl] * scale + (c_chunk + coff)
                pltpu.async_copy(x_ref.at[idxb], rows, sem).wait()
                pltpu.sync_copy(rows,
                                out_ref.at[pl.ds(c_chunk * n_out + r0, kb), :])
                return carry

            lax.fori_loop(0, nblk, blk, 0)

        if active == 32:
            run()
        else:
            pl.when(w < active)(run)

    return pl.kernel(
        body,
        out_type=jax.ShapeDtypeStruct((total, R), _f32),
        mesh=mesh,
        compiler_params=pltpu.CompilerParams(use_tc_tiling_on_sc=False),
        scratch_types=[
            pltpu.VMEM((kb,), jnp.int32),
            pltpu.VMEM((kb, R), _f32),
            pltpu.SemaphoreType.DMA,
        ],
    )


def _gather_rows(x, gidx, n_out, C=1, scale=1, coff=0):
    n_src, R = x.shape
    return _make_gather(n_src, R, n_out, C, scale, coff)(x, gidx)


def _tc_dinv(degS, n):
    R = degS.shape[1]
    bn = min(n, 2048)
    nb = n // bn

    def body(a_ref, b_ref, o_ref):
        d = a_ref[...] + b_ref[...]
        o_ref[...] = 1.0 / jnp.sqrt(jnp.maximum(d[:, :1], 1.0))

    return pl.pallas_call(
        body,
        grid=(nb,),
        in_specs=[pl.BlockSpec((bn, R), lambda i: (i, 0)),
                  pl.BlockSpec((bn, R), lambda i: (i + nb, 0))],
        out_specs=pl.BlockSpec((bn, 1), lambda i: (i, 0)),
        out_shape=jax.ShapeDtypeStruct((n, 1), _f32),
    )(degS, degS)


def _tc_scale(x, dinv, split=False):
    rows, R = x.shape
    n = dinv.shape[0]
    bn = min(n, 512)
    nb = n // bn
    Rh = R // 2

    def body(x_ref, d_ref, *o_refs):
        u = x_ref[...] * d_ref[...]
        if split:
            o_refs[0][...] = u[:, :Rh]
            o_refs[1][...] = u[:, Rh:]
        else:
            o_refs[0][...] = u

    if split:
        out_specs = [pl.BlockSpec((bn, Rh), lambda i: (i, 0))] * 2
        out_shape = [jax.ShapeDtypeStruct((rows, Rh), _f32)] * 2
    else:
        out_specs = [pl.BlockSpec((bn, R), lambda i: (i, 0))]
        out_shape = [jax.ShapeDtypeStruct((rows, R), _f32)]
    res = pl.pallas_call(
        body,
        grid=(rows // bn,),
        in_specs=[pl.BlockSpec((bn, R), lambda i: (i, 0)),
                  pl.BlockSpec((bn, 1), lambda i: (i % nb, 0))],
        out_specs=out_specs,
        out_shape=out_shape,
    )(x, dinv)
    return res if split else res[0]


def _tc_step2(SA, SB, Txm2, dinv, coef, want_u):
    n = dinv.shape[0]
    Rh = SA.shape[1]
    bn = min(n, 512)
    nb = n // bn

    def body(*refs):
        sa1, sa2, sb1, sb2 = refs[0], refs[1], refs[2], refs[3]
        i = 4
        t_ref = refs[i] if Txm2 is not None else None
        if Txm2 is not None:
            i += 1
        d_ref = refs[i]; i += 1
        outs = refs[i:]
        d = d_ref[...]
        txl = coef * d * (sa1[...] + sa2[...])
        txr = coef * d * (sb1[...] + sb2[...])
        if t_ref is not None:
            t = t_ref[...]
            txl = txl - t[:, :Rh]
            txr = txr - t[:, Rh:]
        outs[0][...] = jnp.concatenate([txl, txr], axis=1)
        if want_u:
            outs[1][...] = d * txl
            outs[2][...] = d * txr

    in_specs = [pl.BlockSpec((bn, Rh), lambda i: (i, 0)),
                pl.BlockSpec((bn, Rh), lambda i: (i + nb, 0)),
                pl.BlockSpec((bn, Rh), lambda i: (i, 0)),
                pl.BlockSpec((bn, Rh), lambda i: (i + nb, 0))]
    args = [SA, SA, SB, SB]
    if Txm2 is not None:
        in_specs.append(pl.BlockSpec((bn, 2 * Rh), lambda i: (i, 0)))
        args.append(Txm2)
    in_specs.append(pl.BlockSpec((bn, 1), lambda i: (i, 0)))
    args.append(dinv)

    out_specs = [pl.BlockSpec((bn, 2 * Rh), lambda i: (i, 0))]
    out_shape = [jax.ShapeDtypeStruct((n, 2 * Rh), _f32)]
    if want_u:
        out_specs += [pl.BlockSpec((bn, Rh), lambda i: (i, 0))] * 2
        out_shape += [jax.ShapeDtypeStruct((n, Rh), _f32)] * 2

    res = pl.pallas_call(
        body,
        grid=(nb,),
        in_specs=in_specs,
        out_specs=out_specs,
        out_shape=out_shape,
    )(*args)
    return res if want_u else (res[0], None, None)


def _tc_step(S, Txm2, dinv, coef, want_u, two_slabs, V=None):
    n = dinv.shape[0]
    R = S.shape[1]
    rows = S.shape[0] // 2 if two_slabs else S.shape[0]
    bn = min(n, 512)
    nb = n // bn
    nblocks = rows // bn

    def body(*refs):
        i = 0
        sa_ref = refs[i]; i += 1
        sb_ref = refs[i] if two_slabs else None
        if two_slabs:
            i += 1
        t_ref = refs[i] if Txm2 is not None else None
        if Txm2 is not None:
            i += 1
        v_ref = refs[i] if V is not None else None
        if V is not None:
            i += 1
        d_ref = refs[i]; i += 1
        outs = refs[i:]
        stot = sa_ref[...] + sb_ref[...] if two_slabs else sa_ref[...]
        tx = coef * d_ref[...] * stot
        if t_ref is not None:
            tx = tx - t_ref[...]
        if v_ref is not None:
            tx = tx + v_ref[...]
        outs[0][...] = tx
        if want_u:
            outs[1][...] = d_ref[...] * tx

    in_specs = [pl.BlockSpec((bn, R), lambda i: (i, 0))]
    args = [S]
    if two_slabs:
        in_specs.append(pl.BlockSpec((bn, R), lambda i: (i + nblocks, 0)))
        args.append(S)
    if Txm2 is not None:
        in_specs.append(pl.BlockSpec((bn, R), lambda i: (i, 0)))
        args.append(Txm2)
    if V is not None:
        in_specs.append(pl.BlockSpec((bn, R), lambda i: (i, 0)))
        args.append(V)
    in_specs.append(pl.BlockSpec((bn, 1), lambda i: (i % nb, 0)))
    args.append(dinv)

    out_shape = [jax.ShapeDtypeStruct((rows, R), _f32)]
    out_specs = [pl.BlockSpec((bn, R), lambda i: (i, 0))]
    if want_u:
        out_shape.append(jax.ShapeDtypeStruct((rows, R), _f32))
        out_specs.append(pl.BlockSpec((bn, R), lambda i: (i, 0)))

    res = pl.pallas_call(
        body,
        grid=(nblocks,),
        in_specs=in_specs,
        out_specs=out_specs,
        out_shape=out_shape,
    )(*args)
    return res if want_u else (res[0], None)


def _tc_combine(txs, Wbig, bvec, relu, W2big=None, emit_main=True):
    M, G = txs[0].shape
    Gout = Wbig.shape[2]
    bm = min(M, 512)

    def body(*refs):
        t_refs = refs[:6]
        w_ref, b_ref = refs[6], refs[7]
        i = 8
        w2_ref = refs[i] if W2big is not None else None
        if W2big is not None:
            i += 1
        outs = refs[i:]
        acc = jnp.dot(t_refs[0][...], w_ref[0], preferred_element_type=_f32)
        for k in range(1, 6):
            acc = acc + jnp.dot(t_refs[k][...], w_ref[k],
                                preferred_element_type=_f32)
        acc = acc + b_ref[...]
        if relu:
            acc = jnp.maximum(acc, 0.0)
        j = 0
        if emit_main:
            outs[j][...] = acc
            j += 1
        if W2big is not None:
            outs[j][...] = jnp.dot(acc, w2_ref[...],
                                   preferred_element_type=_f32)

    in_specs = [pl.BlockSpec((bm, G), lambda i: (i, 0)) for _ in range(6)]
    in_specs.append(pl.BlockSpec((6, G, Gout), lambda i: (0, 0, 0)))
    in_specs.append(pl.BlockSpec((1, Gout), lambda i: (0, 0)))
    args = list(txs) + [Wbig, bvec]
    out_specs, out_shape = [], []
    if emit_main:
        out_specs.append(pl.BlockSpec((bm, Gout), lambda i: (i, 0)))
        out_shape.append(jax.ShapeDtypeStruct((M, Gout), _f32))
    if W2big is not None:
        g2 = W2big.shape[1]
        in_specs.append(pl.BlockSpec((Gout, g2), lambda i: (0, 0)))
        args.append(W2big)
        out_specs.append(pl.BlockSpec((bm, g2), lambda i: (i, 0)))
        out_shape.append(jax.ShapeDtypeStruct((M, g2), _f32))

    res = pl.pallas_call(
        body,
        grid=(M // bm,),
        in_specs=in_specs,
        out_specs=out_specs,
        out_shape=out_shape,
    )(*args)
    return res[0] if len(res) == 1 else res


def _tc_latent(h, We3, be, Wd3, bd3):
    def body(h_ref, we_ref, be_ref, wd_ref, bd_ref, o_ref):
        acc = jnp.broadcast_to(be_ref[...], (_B, _Z))
        for nn in range(_Z):
            acc = acc + jnp.dot(h_ref[nn], we_ref[nn],
                                preferred_element_type=_f32)
        z = jnp.maximum(acc, 0.0)
        for nn in range(_Z):
            o_ref[nn] = jnp.maximum(
                jnp.dot(z, wd_ref[nn], preferred_element_type=_f32)
                + bd_ref[nn], 0.0)

    return pl.pallas_call(
        body,
        out_shape=jax.ShapeDtypeStruct((_Z, _B, 32), _f32),
    )(h, We3, be.reshape(1, _Z), Wd3, bd3)


def _cheb_layer(h_flat, edge, dinv1, W, b, relu, n, C, W2=None):
    E = edge.shape[1]
    R = h_flat.shape[1]
    f_in = W.shape[1]
    two = C == 1
    split = C == 1 and n * R >= 1 << 20

    if split:
        Rh = R // 2
        ua, ub = _tc_scale(h_flat, dinv1, split=True)
        SA = _spmv(ua, edge, n, Rh, E, 1)
        SB = _spmv(ub, edge, n, Rh, E, 1)
        Tx1, ua, ub = _tc_step2(SA, SB, None, dinv1, -1.0, True)
        txs = [h_flat, Tx1]
        for k in range(2, _K):
            SA = _spmv(ua, edge, n, Rh, E, 1)
            SB = _spmv(ub, edge, n, Rh, E, 1)
            Txk, ua, ub = _tc_step2(SA, SB, txs[k - 2], dinv1, -2.0,
                                    k < _K - 1)
            txs.append(Txk)
    else:
        u0 = _tc_scale(h_flat, dinv1)
        S0 = _spmv(u0, edge, n, R, E, C)
        Tx1, u = _tc_step(S0, None, dinv1, -1.0, True, two)
        txs = [h_flat, Tx1]
        for k in range(2, _K):
            Sk = _spmv(u, edge, n, R, E, C)
            Txk, u = _tc_step(Sk, txs[k - 2], dinv1, -2.0, k < _K - 1, two)
            txs.append(Txk)

    g = W.shape[2]
    bc = R // f_in
    eye = jnp.eye(bc, dtype=_f32)
    Wbig = (eye[None, :, None, :, None] *
            W[:, None, :, None, :]).reshape(_K, bc * f_in, bc * g)
    bb = jnp.zeros((g,), _f32) if b is None else b
    bvec = jnp.tile(bb, bc).reshape(1, bc * g)
    return _tc_combine(txs, Wbig, bvec, relu, W2big=W2,
                       emit_main=(W2 is None))


def _tc_clenstep(S, bm2, dinv, coef, want_u, Vs, k, C):
    n = dinv.shape[0]
    w = Vs[0].shape[1] // 6
    R = w * C * len(Vs)
    bn = min(n, 512)
    nb_ = n // bn

    def body(*refs):
        i = 0
        if S is not None:
            sa, sb = refs[0], refs[1]
            i = 2
        t_ref = refs[i] if bm2 is not None else None
        if bm2 is not None:
            i += 1
        nv = C * len(Vs)
        v_refs = refs[i:i + nv]; i += nv
        d_ref = refs[i]; i += 1
        outs = refs[i:]
        v = jnp.concatenate(
            [vr[:, w * k:w * (k + 1)] for vr in v_refs], axis=1)
        d = d_ref[...]
        if S is not None:
            b_ = coef * d * (sa[...] + sb[...]) + v
        else:
            b_ = v
        if t_ref is not None:
            b_ = b_ - t_ref[...]
        outs[0][...] = b_
        if want_u:
            outs[1][...] = d * b_

    in_specs, args = [], []
    if S is not None:
        in_specs += [pl.BlockSpec((bn, R), lambda i: (i, 0)),
                     pl.BlockSpec((bn, R), lambda i: (i + nb_, 0))]
        args += [S, S]
    if bm2 is not None:
        in_specs.append(pl.BlockSpec((bn, R), lambda i: (i, 0)))
        args.append(bm2)
    for V in Vs:
        for c in range(C):
            in_specs.append(
                pl.BlockSpec((bn, 6 * w), lambda i, c=c: (i + c * nb_, 0)))
            args.append(V)
    in_specs.append(pl.BlockSpec((bn, 1), lambda i: (i, 0)))
    args.append(dinv)
    out_specs = [pl.BlockSpec((bn, R), lambda i: (i, 0))]
    out_shape = [jax.ShapeDtypeStruct((n, R), _f32)]
    if want_u:
        out_specs.append(pl.BlockSpec((bn, R), lambda i: (i, 0)))
        out_shape.append(jax.ShapeDtypeStruct((n, R), _f32))
    res = pl.pallas_call(
        body,
        grid=(n // bn,),
        in_specs=in_specs,
        out_specs=out_specs,
        out_shape=out_shape,
    )(*args)
    return res if want_u else (res[0], None)


def _cheb_clenshaw(Vs, edge, dinv1, n, C):
    E = edge.shape[1]
    R = (Vs[0].shape[1] // 6) * C * len(Vs)
    b_k1, u = _tc_clenstep(None, None, dinv1, 0.0, True, Vs, 5, C)
    b_k2 = None
    for k in range(4, 0, -1):
        S = _spmv(u, edge, n, R, E, 1)
        b_k, u = _tc_clenstep(S, b_k2, dinv1, -2.0, True, Vs, k, C)
        b_k2, b_k1 = b_k1, b_k
    S = _spmv(u, edge, n, R, E, 1)
    out, _ = _tc_clenstep(S, b_k2, dinv1, -1.0, False, Vs, 0, C)
    return out


def _level_dinv(edge, n):
    E = edge.shape[1]
    R = 48 if n == _N[0] else 16
    ones = jnp.ones((n, R), _f32)
    degS = _spmv(ones, edge, n, R, E)
    return _tc_dinv(degS, n)


def kernel(x, edges, down_idx, up_idx, enc_W, enc_b, dec_W, dec_b,
           lin_enc_W, lin_enc_b, lin_dec_W, lin_dec_b):
    n0 = _N[0]
    dinvs = [None] * 4

    def dinv_for(lvl):
        if dinvs[lvl] is None:
            dinvs[lvl] = _level_dinv(edges[lvl], _N[lvl])
        return dinvs[lvl]

    h = x.reshape(_B, n0, _ENC_F[0]).transpose(1, 0, 2).reshape(
        n0, _B * _ENC_F[0])
    for i in range(4):
        out = _cheb_layer(h, edges[i], dinv_for(i), enc_W[i], enc_b[i],
                          True, _N[i], 1)
        h = _gather_rows(out, down_idx[i], _N[i + 1])

    We3 = lin_enc_W.reshape(_Z, 32, _Z)
    Wd3 = lin_dec_W.reshape(_Z, _Z, 32).swapaxes(0, 1)
    bd3 = lin_dec_b.reshape(_Z, 1, 32)
    h = _tc_latent(h.reshape(_Z, _B, 32), We3, lin_enc_b, Wd3, bd3)
    h = h.reshape(_Z, _B * 32)

    for i in range(4):
        lvl = 3 - i
        n = _N[lvl]
        f_in, f_out = _DEC[i]
        if lvl > 0:
            hu = _gather_rows(h, up_idx[lvl], n)
            h = _cheb_layer(hu, edges[lvl], dinv_for(lvl), dec_W[i],
                            dec_b[i], True, n, 1)
        else:
            C, bc = 4, 2
            hsrc = h.reshape(_N[1] * 2 * C, bc * f_in)
            huA = _gather_rows(hsrc, up_idx[0], n, C=C, scale=2 * C, coff=0)
            huB = _gather_rows(hsrc, up_idx[0], n, C=C, scale=2 * C, coff=C)
            Wf = dec_W[4]
            W2big = (jnp.eye(bc, dtype=_f32)[:, None, None, :, None] *
                     Wf.transpose(1, 0, 2)[None, :, :, None, :]).reshape(
                         bc * 16, _K * bc * _ENC_F[0])
            VA = _cheb_layer(huA, edges[0], dinv_for(0), dec_W[i],
                             dec_b[i], True, n, C, W2=W2big)
            VB = _cheb_layer(huB, edges[0], dinv_for(0), dec_W[i],
                             dec_b[i], True, n, C, W2=W2big)
            res = _cheb_clenshaw([VA, VB], edges[0], dinv_for(0), n, C)
            return res.reshape(n, _B, _ENC_F[0]).transpose(1, 0, 2).reshape(
                _B * n, _ENC_F[0])


# speedup vs baseline: 1.1052x; 1.1052x over previous
import functools

import jax
import jax.numpy as jnp
from jax import lax
from jax.experimental import pallas as pl
from jax.experimental.pallas import tpu as pltpu
from jax.experimental.pallas import tpu_sc as plsc

_N = [16384, 4096, 1024, 256, 64]
_ENC_F = [3, 16, 16, 16, 32]
_DEC = [(32, 16), (16, 16), (16, 16), (16, 16), (16, 3)]
_K = 6
_Z = 64
_B = 16
_KB = 128

_f32 = jnp.float32


_NB = 4
_WIN = 32


@functools.lru_cache(None)
def _make_spmv(n, R, E, C):
    mesh = plsc.VectorSubcoreMesh(core_axis_name="c", subcore_axis_name="s")
    n16 = n // 16
    Cout = 2 if C == 1 else C
    ept = E // 32 if C == 1 else E // 16
    KB = 128 if R <= 64 else (64 if R <= 256 else 32)
    KB = min(KB, ept)
    nblk = ept // KB
    assert ept % KB == 0 and n % 16 == 0

    def body(x_ref, edge_ref, z_ref, out_ref, *scr):
        eidx = scr[0:_NB]
        rows = scr[_NB:2 * _NB]
        isems = scr[2 * _NB:3 * _NB]
        gsems = scr[3 * _NB:4 * _NB]
        ssems = scr[4 * _NB:5 * _NB]
        acc = scr[5 * _NB]
        c = lax.axis_index("c")
        s = lax.axis_index("s")

        def pipe_window(xoff, base, W):
            idesc, gdesc, sdesc = {}, {}, {}
            for j in range(W + 2):
                if j < W:
                    b = j % _NB
                    if j >= _NB:
                        sdesc[j - _NB].wait()
                    idesc[j] = pltpu.async_copy(
                        edge_ref.at[:, pl.ds(base + j * KB, KB)],
                        eidx[b], isems[b])
                jg = j - 1
                if 0 <= jg < W:
                    bg = jg % _NB
                    idesc[jg].wait()
                    if C > 1:
                        for v in range(KB // 16):
                            sl = pl.ds(v * 16, 16)
                            eidx[bg][0, sl] = eidx[bg][0, sl] + xoff
                    gdesc[jg] = pltpu.async_copy(
                        x_ref.at[eidx[bg].at[0]], rows[bg], gsems[bg])
                js = j - 2
                if 0 <= js < W:
                    bs = js % _NB
                    gdesc[js].wait()
                    sdesc[js] = pltpu.async_copy(
                        rows[bs], acc.at[eidx[bs].at[1]], ssems[bs], add=True)
            for j in range(max(0, W - _NB), W):
                sdesc[j].wait()

        for jc in range(C // 2 if C > 1 else 1):
            p = (2 * jc + c) if C > 1 else 0
            pltpu.sync_copy(z_ref, acc.at[pl.ds(s * n16, n16), :])
            plsc.subcore_barrier()
            if C == 1:
                e_base = c * (E // 2) + s * ept
            else:
                e_base = s * ept
            xoff = p * n
            if nblk <= _WIN:
                pipe_window(xoff, e_base, nblk)
            else:
                def outer(w, carry, xoff=xoff, e_base=e_base):
                    pipe_window(xoff, e_base + w * (_WIN * KB), _WIN)
                    return carry
                lax.fori_loop(0, nblk // _WIN, outer, 0)
            plsc.subcore_barrier()
            obase = (c if C == 1 else p) * n
            pltpu.sync_copy(acc.at[pl.ds(s * n16, n16), :],
                            out_ref.at[pl.ds(obase + s * n16, n16), :])

    assert nblk <= _WIN or nblk % _WIN == 0
    scratch = ([pltpu.VMEM((2, KB), jnp.int32) for _ in range(_NB)]
               + [pltpu.VMEM((KB, R), _f32) for _ in range(_NB)]
               + [pltpu.SemaphoreType.DMA for _ in range(3 * _NB)]
               + [pltpu.VMEM_SHARED((n, R), _f32)])
    return pl.kernel(
        body,
        out_type=jax.ShapeDtypeStruct((Cout * n, R), _f32),
        mesh=mesh,
        compiler_params=pltpu.CompilerParams(use_tc_tiling_on_sc=False),
        scratch_types=scratch,
    )


def _spmv(x, edge, n, R, E, C=1):
    zeros = jnp.zeros((n // 16, R), _f32)
    return _make_spmv(n, R, E, C)(x, edge, zeros)


@functools.lru_cache(None)
def _make_gather(n_src, R, n_out, C, scale, coff):
    mesh = plsc.VectorSubcoreMesh(core_axis_name="c", subcore_axis_name="s")
    total = C * n_out
    m = max(total // 32, 8)
    active = total // m
    kb = min(_KB, m)
    nblk = m // kb
    tpc = active // C

    def body(x_ref, gidx_ref, out_ref, idxb, rows, sem):
        c = lax.axis_index("c")
        s = lax.axis_index("s")
        w = s * 2 + c

        def run():
            c_chunk = w // tpc
            i0 = (w % tpc) * m

            def blk(j, carry):
                r0 = i0 + j * kb
                pltpu.sync_copy(gidx_ref.at[pl.ds(r0, kb)], idxb)
                if scale != 1 or C > 1 or coff:
                    for v in range(kb // 16):
                        sl = pl.ds(v * 16, 16)
                        idxb[sl] = idxb[sl] * scale + (c_chunk + coff)
                pltpu.async_copy(x_ref.at[idxb], rows, sem).wait()
                pltpu.sync_copy(rows,
                                out_ref.at[pl.ds(c_chunk * n_out + r0, kb), :])
                return carry

            lax.fori_loop(0, nblk, blk, 0)

        if active == 32:
            run()
        else:
            pl.when(w < active)(run)

    return pl.kernel(
        body,
        out_type=jax.ShapeDtypeStruct((total, R), _f32),
        mesh=mesh,
        compiler_params=pltpu.CompilerParams(use_tc_tiling_on_sc=False),
        scratch_types=[
            pltpu.VMEM((kb,), jnp.int32),
            pltpu.VMEM((kb, R), _f32),
            pltpu.SemaphoreType.DMA,
        ],
    )


def _gather_rows(x, gidx, n_out, C=1, scale=1, coff=0):
    n_src, R = x.shape
    return _make_gather(n_src, R, n_out, C, scale, coff)(x, gidx)


def _tc_dinv(degS, n):
    R = degS.shape[1]
    bn = min(n, 2048)
    nb = n // bn

    def body(a_ref, b_ref, o_ref):
        d = a_ref[...] + b_ref[...]
        o_ref[...] = 1.0 / jnp.sqrt(jnp.maximum(d[:, :1], 1.0))

    return pl.pallas_call(
        body,
        grid=(nb,),
        in_specs=[pl.BlockSpec((bn, R), lambda i: (i, 0)),
                  pl.BlockSpec((bn, R), lambda i: (i + nb, 0))],
        out_specs=pl.BlockSpec((bn, 1), lambda i: (i, 0)),
        out_shape=jax.ShapeDtypeStruct((n, 1), _f32),
    )(degS, degS)


def _tc_scale(x, dinv, split=False):
    rows, R = x.shape
    n = dinv.shape[0]
    bn = min(n, 512)
    nb = n // bn
    Rh = R // 2

    def body(x_ref, d_ref, *o_refs):
        u = x_ref[...] * d_ref[...]
        if split:
            o_refs[0][...] = u[:, :Rh]
            o_refs[1][...] = u[:, Rh:]
        else:
            o_refs[0][...] = u

    if split:
        out_specs = [pl.BlockSpec((bn, Rh), lambda i: (i, 0))] * 2
        out_shape = [jax.ShapeDtypeStruct((rows, Rh), _f32)] * 2
    else:
        out_specs = [pl.BlockSpec((bn, R), lambda i: (i, 0))]
        out_shape = [jax.ShapeDtypeStruct((rows, R), _f32)]
    res = pl.pallas_call(
        body,
        grid=(rows // bn,),
        in_specs=[pl.BlockSpec((bn, R), lambda i: (i, 0)),
                  pl.BlockSpec((bn, 1), lambda i: (i % nb, 0))],
        out_specs=out_specs,
        out_shape=out_shape,
    )(x, dinv)
    return res if split else res[0]


def _tc_step2(SA, SB, Txm2, dinv, coef, want_u):
    n = dinv.shape[0]
    Rh = SA.shape[1]
    bn = min(n, 512)
    nb = n // bn

    def body(*refs):
        sa1, sa2, sb1, sb2 = refs[0], refs[1], refs[2], refs[3]
        i = 4
        t_ref = refs[i] if Txm2 is not None else None
        if Txm2 is not None:
            i += 1
        d_ref = refs[i]; i += 1
        outs = refs[i:]
        d = d_ref[...]
        txl = coef * d * (sa1[...] + sa2[...])
        txr = coef * d * (sb1[...] + sb2[...])
        if t_ref is not None:
            t = t_ref[...]
            txl = txl - t[:, :Rh]
            txr = txr - t[:, Rh:]
        outs[0][...] = jnp.concatenate([txl, txr], axis=1)
        if want_u:
            outs[1][...] = d * txl
            outs[2][...] = d * txr

    in_specs = [pl.BlockSpec((bn, Rh), lambda i: (i, 0)),
                pl.BlockSpec((bn, Rh), lambda i: (i + nb, 0)),
                pl.BlockSpec((bn, Rh), lambda i: (i, 0)),
                pl.BlockSpec((bn, Rh), lambda i: (i + nb, 0))]
    args = [SA, SA, SB, SB]
    if Txm2 is not None:
        in_specs.append(pl.BlockSpec((bn, 2 * Rh), lambda i: (i, 0)))
        args.append(Txm2)
    in_specs.append(pl.BlockSpec((bn, 1), lambda i: (i, 0)))
    args.append(dinv)

    out_specs = [pl.BlockSpec((bn, 2 * Rh), lambda i: (i, 0))]
    out_shape = [jax.ShapeDtypeStruct((n, 2 * Rh), _f32)]
    if want_u:
        out_specs += [pl.BlockSpec((bn, Rh), lambda i: (i, 0))] * 2
        out_shape += [jax.ShapeDtypeStruct((n, Rh), _f32)] * 2

    res = pl.pallas_call(
        body,
        grid=(nb,),
        in_specs=in_specs,
        out_specs=out_specs,
        out_shape=out_shape,
    )(*args)
    return res if want_u else (res[0], None, None)


def _tc_step(S, Txm2, dinv, coef, want_u, two_slabs, V=None):
    n = dinv.shape[0]
    R = S.shape[1]
    rows = S.shape[0] // 2 if two_slabs else S.shape[0]
    bn = min(n, 512)
    nb = n // bn
    nblocks = rows // bn

    def body(*refs):
        i = 0
        sa_ref = refs[i]; i += 1
        sb_ref = refs[i] if two_slabs else None
        if two_slabs:
            i += 1
        t_ref = refs[i] if Txm2 is not None else None
        if Txm2 is not None:
            i += 1
        v_ref = refs[i] if V is not None else None
        if V is not None:
            i += 1
        d_ref = refs[i]; i += 1
        outs = refs[i:]
        stot = sa_ref[...] + sb_ref[...] if two_slabs else sa_ref[...]
        tx = coef * d_ref[...] * stot
        if t_ref is not None:
            tx = tx - t_ref[...]
        if v_ref is not None:
            tx = tx + v_ref[...]
        outs[0][...] = tx
        if want_u:
            outs[1][...] = d_ref[...] * tx

    in_specs = [pl.BlockSpec((bn, R), lambda i: (i, 0))]
    args = [S]
    if two_slabs:
        in_specs.append(pl.BlockSpec((bn, R), lambda i: (i + nblocks, 0)))
        args.append(S)
    if Txm2 is not None:
        in_specs.append(pl.BlockSpec((bn, R), lambda i: (i, 0)))
        args.append(Txm2)
    if V is not None:
        in_specs.append(pl.BlockSpec((bn, R), lambda i: (i, 0)))
        args.append(V)
    in_specs.append(pl.BlockSpec((bn, 1), lambda i: (i % nb, 0)))
    args.append(dinv)

    out_shape = [jax.ShapeDtypeStruct((rows, R), _f32)]
    out_specs = [pl.BlockSpec((bn, R), lambda i: (i, 0))]
    if want_u:
        out_shape.append(jax.ShapeDtypeStruct((rows, R), _f32))
        out_specs.append(pl.BlockSpec((bn, R), lambda i: (i, 0)))

    res = pl.pallas_call(
        body,
        grid=(nblocks,),
        in_specs=in_specs,
        out_specs=out_specs,
        out_shape=out_shape,
    )(*args)
    return res if want_u else (res[0], None)


def _tc_combine(txs, Wbig, bvec, relu, W2big=None, emit_main=True):
    M, G = txs[0].shape
    Gout = Wbig.shape[2]
    bm = min(M, 512)

    def body(*refs):
        t_refs = refs[:6]
        w_ref, b_ref = refs[6], refs[7]
        i = 8
        w2_ref = refs[i] if W2big is not None else None
        if W2big is not None:
            i += 1
        outs = refs[i:]
        acc = jnp.dot(t_refs[0][...], w_ref[0], preferred_element_type=_f32)
        for k in range(1, 6):
            acc = acc + jnp.dot(t_refs[k][...], w_ref[k],
                                preferred_element_type=_f32)
        acc = acc + b_ref[...]
        if relu:
            acc = jnp.maximum(acc, 0.0)
        j = 0
        if emit_main:
            outs[j][...] = acc
            j += 1
        if W2big is not None:
            outs[j][...] = jnp.dot(acc, w2_ref[...],
                                   preferred_element_type=_f32)

    in_specs = [pl.BlockSpec((bm, G), lambda i: (i, 0)) for _ in range(6)]
    in_specs.append(pl.BlockSpec((6, G, Gout), lambda i: (0, 0, 0)))
    in_specs.append(pl.BlockSpec((1, Gout), lambda i: (0, 0)))
    args = list(txs) + [Wbig, bvec]
    out_specs, out_shape = [], []
    if emit_main:
        out_specs.append(pl.BlockSpec((bm, Gout), lambda i: (i, 0)))
        out_shape.append(jax.ShapeDtypeStruct((M, Gout), _f32))
    if W2big is not None:
        g2 = W2big.shape[1]
        in_specs.append(pl.BlockSpec((Gout, g2), lambda i: (0, 0)))
        args.append(W2big)
        out_specs.append(pl.BlockSpec((bm, g2), lambda i: (i, 0)))
        out_shape.append(jax.ShapeDtypeStruct((M, g2), _f32))

    res = pl.pallas_call(
        body,
        grid=(M // bm,),
        in_specs=in_specs,
        out_specs=out_specs,
        out_shape=out_shape,
    )(*args)
    return res[0] if len(res) == 1 else res


def _tc_latent(h, We3, be, Wd3, bd3):
    def body(h_ref, we_ref, be_ref, wd_ref, bd_ref, o_ref):
        acc = jnp.broadcast_to(be_ref[...], (_B, _Z))
        for nn in range(_Z):
            acc = acc + jnp.dot(h_ref[nn], we_ref[nn],
                                preferred_element_type=_f32)
        z = jnp.maximum(acc, 0.0)
        for nn in range(_Z):
            o_ref[nn] = jnp.maximum(
                jnp.dot(z, wd_ref[nn], preferred_element_type=_f32)
                + bd_ref[nn], 0.0)

    return pl.pallas_call(
        body,
        out_shape=jax.ShapeDtypeStruct((_Z, _B, 32), _f32),
    )(h, We3, be.reshape(1, _Z), Wd3, bd3)


def _cheb_layer(h_flat, edge, dinv1, W, b, relu, n, C, W2=None):
    E = edge.shape[1]
    R = h_flat.shape[1]
    f_in = W.shape[1]
    two = C == 1
    split = C == 1 and n * R >= 1 << 20

    if split:
        Rh = R // 2
        ua, ub = _tc_scale(h_flat, dinv1, split=True)
        SA = _spmv(ua, edge, n, Rh, E, 1)
        SB = _spmv(ub, edge, n, Rh, E, 1)
        Tx1, ua, ub = _tc_step2(SA, SB, None, dinv1, -1.0, True)
        txs = [h_flat, Tx1]
        for k in range(2, _K):
            SA = _spmv(ua, edge, n, Rh, E, 1)
            SB = _spmv(ub, edge, n, Rh, E, 1)
            Txk, ua, ub = _tc_step2(SA, SB, txs[k - 2], dinv1, -2.0,
                                    k < _K - 1)
            txs.append(Txk)
    else:
        u0 = _tc_scale(h_flat, dinv1)
        S0 = _spmv(u0, edge, n, R, E, C)
        Tx1, u = _tc_step(S0, None, dinv1, -1.0, True, two)
        txs = [h_flat, Tx1]
        for k in range(2, _K):
            Sk = _spmv(u, edge, n, R, E, C)
            Txk, u = _tc_step(Sk, txs[k - 2], dinv1, -2.0, k < _K - 1, two)
            txs.append(Txk)

    g = W.shape[2]
    bc = R // f_in
    eye = jnp.eye(bc, dtype=_f32)
    Wbig = (eye[None, :, None, :, None] *
            W[:, None, :, None, :]).reshape(_K, bc * f_in, bc * g)
    bb = jnp.zeros((g,), _f32) if b is None else b
    bvec = jnp.tile(bb, bc).reshape(1, bc * g)
    return _tc_combine(txs, Wbig, bvec, relu, W2big=W2,
                       emit_main=(W2 is None))


def _tc_clenstep(S, bm2, dinv, coef, want_u, Vs, k, C):
    n = dinv.shape[0]
    w = Vs[0].shape[1] // 6
    R = w * C * len(Vs)
    bn = min(n, 512)
    nb_ = n // bn

    def body(*refs):
        i = 0
        if S is not None:
            sa, sb = refs[0], refs[1]
            i = 2
        t_ref = refs[i] if bm2 is not None else None
        if bm2 is not None:
            i += 1
        nv = C * len(Vs)
        v_refs = refs[i:i + nv]; i += nv
        d_ref = refs[i]; i += 1
        outs = refs[i:]
        v = jnp.concatenate(
            [vr[:, w * k:w * (k + 1)] for vr in v_refs], axis=1)
        d = d_ref[...]
        if S is not None:
            b_ = coef * d * (sa[...] + sb[...]) + v
        else:
            b_ = v
        if t_ref is not None:
            b_ = b_ - t_ref[...]
        outs[0][...] = b_
        if want_u:
            outs[1][...] = d * b_

    in_specs, args = [], []
    if S is not None:
        in_specs += [pl.BlockSpec((bn, R), lambda i: (i, 0)),
                     pl.BlockSpec((bn, R), lambda i: (i + nb_, 0))]
        args += [S, S]
    if bm2 is not None:
        in_specs.append(pl.BlockSpec((bn, R), lambda i: (i, 0)))
        args.append(bm2)
    for V in Vs:
        for c in range(C):
            in_specs.append(
                pl.BlockSpec((bn, 6 * w), lambda i, c=c: (i + c * nb_, 0)))
            args.append(V)
    in_specs.append(pl.BlockSpec((bn, 1), lambda i: (i, 0)))
    args.append(dinv)
    out_specs = [pl.BlockSpec((bn, R), lambda i: (i, 0))]
    out_shape = [jax.ShapeDtypeStruct((n, R), _f32)]
    if want_u:
        out_specs.append(pl.BlockSpec((bn, R), lambda i: (i, 0)))
        out_shape.append(jax.ShapeDtypeStruct((n, R), _f32))
    res = pl.pallas_call(
        body,
        grid=(n // bn,),
        in_specs=in_specs,
        out_specs=out_specs,
        out_shape=out_shape,
    )(*args)
    return res if want_u else (res[0], None)


def _cheb_clenshaw(Vs, edge, dinv1, n, C):
    E = edge.shape[1]
    R = (Vs[0].shape[1] // 6) * C * len(Vs)
    b_k1, u = _tc_clenstep(None, None, dinv1, 0.0, True, Vs, 5, C)
    b_k2 = None
    for k in range(4, 0, -1):
        S = _spmv(u, edge, n, R, E, 1)
        b_k, u = _tc_clenstep(S, b_k2, dinv1, -2.0, True, Vs, k, C)
        b_k2, b_k1 = b_k1, b_k
    S = _spmv(u, edge, n, R, E, 1)
    out, _ = _tc_clenstep(S, b_k2, dinv1, -1.0, False, Vs, 0, C)
    return out


def _level_dinv(edge, n):
    E = edge.shape[1]
    R = 48 if n == _N[0] else 16
    ones = jnp.ones((n, R), _f32)
    degS = _spmv(ones, edge, n, R, E)
    return _tc_dinv(degS, n)


def kernel(x, edges, down_idx, up_idx, enc_W, enc_b, dec_W, dec_b,
           lin_enc_W, lin_enc_b, lin_dec_W, lin_dec_b):
    n0 = _N[0]
    dinvs = [None] * 4

    def dinv_for(lvl):
        if dinvs[lvl] is None:
            dinvs[lvl] = _level_dinv(edges[lvl], _N[lvl])
        return dinvs[lvl]

    h = x.reshape(_B, n0, _ENC_F[0]).transpose(1, 0, 2).reshape(
        n0, _B * _ENC_F[0])
    for i in range(4):
        out = _cheb_layer(h, edges[i], dinv_for(i), enc_W[i], enc_b[i],
                          True, _N[i], 1)
        h = _gather_rows(out, down_idx[i], _N[i + 1])

    We3 = lin_enc_W.reshape(_Z, 32, _Z)
    Wd3 = lin_dec_W.reshape(_Z, _Z, 32).swapaxes(0, 1)
    bd3 = lin_dec_b.reshape(_Z, 1, 32)
    h = _tc_latent(h.reshape(_Z, _B, 32), We3, lin_enc_b, Wd3, bd3)
    h = h.reshape(_Z, _B * 32)

    for i in range(4):
        lvl = 3 - i
        n = _N[lvl]
        f_in, f_out = _DEC[i]
        if lvl > 0:
            hu = _gather_rows(h, up_idx[lvl], n)
            h = _cheb_layer(hu, edges[lvl], dinv_for(lvl), dec_W[i],
                            dec_b[i], True, n, 1)
        else:
            C, bc = 4, 4
            hu = _gather_rows(h.reshape(_N[1] * C, bc * f_in),
                              up_idx[0], n, C=C, scale=C)
            Wf = dec_W[4]
            W2big = (jnp.eye(bc, dtype=_f32)[:, None, None, :, None] *
                     Wf.transpose(1, 0, 2)[None, :, :, None, :]).reshape(
                         bc * 16, _K * bc * _ENC_F[0])
            V = _cheb_layer(hu, edges[0], dinv_for(0), dec_W[i],
                            dec_b[i], True, n, C, W2=W2big)
            res = _cheb_clenshaw([V], edges[0], dinv_for(0), n, C)
            return res.reshape(n, _B, _ENC_F[0]).transpose(1, 0, 2).reshape(
                _B * n, _ENC_F[0])


# speedup vs baseline: 1.1075x; 1.0020x over previous
import functools

import jax
import jax.numpy as jnp
from jax import lax
from jax.experimental import pallas as pl
from jax.experimental.pallas import tpu as pltpu
from jax.experimental.pallas import tpu_sc as plsc

_N = [16384, 4096, 1024, 256, 64]
_ENC_F = [3, 16, 16, 16, 32]
_DEC = [(32, 16), (16, 16), (16, 16), (16, 16), (16, 3)]
_K = 6
_Z = 64
_B = 16
_KB = 128

_f32 = jnp.float32


_NB = 6
_WIN = 32


@functools.lru_cache(None)
def _make_spmv(n, R, E, C):
    mesh = plsc.VectorSubcoreMesh(core_axis_name="c", subcore_axis_name="s")
    n16 = n // 16
    Cout = 2 if C == 1 else C
    ept = E // 32 if C == 1 else E // 16
    KB = 128 if R <= 64 else (64 if R <= 256 else 32)
    KB = min(KB, ept)
    nblk = ept // KB
    assert ept % KB == 0 and n % 16 == 0

    def body(x_ref, edge_ref, z_ref, out_ref, *scr):
        eidx = scr[0:_NB]
        rows = scr[_NB:2 * _NB]
        isems = scr[2 * _NB:3 * _NB]
        gsems = scr[3 * _NB:4 * _NB]
        ssems = scr[4 * _NB:5 * _NB]
        acc = scr[5 * _NB]
        c = lax.axis_index("c")
        s = lax.axis_index("s")

        def pipe_window(xoff, base, W):
            idesc, gdesc, sdesc = {}, {}, {}
            for j in range(W + 2):
                if j < W:
                    b = j % _NB
                    if j >= _NB:
                        sdesc[j - _NB].wait()
                    idesc[j] = pltpu.async_copy(
                        edge_ref.at[:, pl.ds(base + j * KB, KB)],
                        eidx[b], isems[b])
                jg = j - 1
                if 0 <= jg < W:
                    bg = jg % _NB
                    idesc[jg].wait()
                    if C > 1:
                        for v in range(KB // 16):
                            sl = pl.ds(v * 16, 16)
                            eidx[bg][0, sl] = eidx[bg][0, sl] + xoff
                    gdesc[jg] = pltpu.async_copy(
                        x_ref.at[eidx[bg].at[0]], rows[bg], gsems[bg])
                js = j - 2
                if 0 <= js < W:
                    bs = js % _NB
                    gdesc[js].wait()
                    sdesc[js] = pltpu.async_copy(
                        rows[bs], acc.at[eidx[bs].at[1]], ssems[bs], add=True)
            for j in range(max(0, W - _NB), W):
                sdesc[j].wait()

        for jc in range(C // 2 if C > 1 else 1):
            p = (2 * jc + c) if C > 1 else 0
            pltpu.sync_copy(z_ref, acc.at[pl.ds(s * n16, n16), :])
            plsc.subcore_barrier()
            if C == 1:
                e_base = c * (E // 2) + s * ept
            else:
                e_base = s * ept
            xoff = p * n
            if nblk <= _WIN:
                pipe_window(xoff, e_base, nblk)
            else:
                def outer(w, carry, xoff=xoff, e_base=e_base):
                    pipe_window(xoff, e_base + w * (_WIN * KB), _WIN)
                    return carry
                lax.fori_loop(0, nblk // _WIN, outer, 0)
            plsc.subcore_barrier()
            obase = (c if C == 1 else p) * n
            pltpu.sync_copy(acc.at[pl.ds(s * n16, n16), :],
                            out_ref.at[pl.ds(obase + s * n16, n16), :])

    assert nblk <= _WIN or nblk % _WIN == 0
    scratch = ([pltpu.VMEM((2, KB), jnp.int32) for _ in range(_NB)]
               + [pltpu.VMEM((KB, R), _f32) for _ in range(_NB)]
               + [pltpu.SemaphoreType.DMA for _ in range(3 * _NB)]
               + [pltpu.VMEM_SHARED((n, R), _f32)])
    return pl.kernel(
        body,
        out_type=jax.ShapeDtypeStruct((Cout * n, R), _f32),
        mesh=mesh,
        compiler_params=pltpu.CompilerParams(use_tc_tiling_on_sc=False),
        scratch_types=scratch,
    )


def _spmv(x, edge, n, R, E, C=1):
    zeros = jnp.zeros((n // 16, R), _f32)
    return _make_spmv(n, R, E, C)(x, edge, zeros)


@functools.lru_cache(None)
def _make_gather(n_src, R, n_out, C, scale, coff):
    mesh = plsc.VectorSubcoreMesh(core_axis_name="c", subcore_axis_name="s")
    total = C * n_out
    m = max(total // 32, 8)
    active = total // m
    kb = min(_KB, m)
    nblk = m // kb
    tpc = active // C

    def body(x_ref, gidx_ref, out_ref, idxb, rows, sem):
        c = lax.axis_index("c")
        s = lax.axis_index("s")
        w = s * 2 + c

        def run():
            c_chunk = w // tpc
            i0 = (w % tpc) * m

            def blk(j, carry):
                r0 = i0 + j * kb
                pltpu.sync_copy(gidx_ref.at[pl.ds(r0, kb)], idxb)
                if scale != 1 or C > 1 or coff:
                    for v in range(kb // 16):
                        sl = pl.ds(v * 16, 16)
                        idxb[sl] = idxb[sl] * scale + (c_chunk + coff)
                pltpu.async_copy(x_ref.at[idxb], rows, sem).wait()
                pltpu.sync_copy(rows,
                                out_ref.at[pl.ds(c_chunk * n_out + r0, kb), :])
                return carry

            lax.fori_loop(0, nblk, blk, 0)

        if active == 32:
            run()
        else:
            pl.when(w < active)(run)

    return pl.kernel(
        body,
        out_type=jax.ShapeDtypeStruct((total, R), _f32),
        mesh=mesh,
        compiler_params=pltpu.CompilerParams(use_tc_tiling_on_sc=False),
        scratch_types=[
            pltpu.VMEM((kb,), jnp.int32),
            pltpu.VMEM((kb, R), _f32),
            pltpu.SemaphoreType.DMA,
        ],
    )


def _gather_rows(x, gidx, n_out, C=1, scale=1, coff=0):
    n_src, R = x.shape
    return _make_gather(n_src, R, n_out, C, scale, coff)(x, gidx)


def _tc_dinv(degS, n):
    R = degS.shape[1]
    bn = min(n, 2048)
    nb = n // bn

    def body(a_ref, b_ref, o_ref):
        d = a_ref[...] + b_ref[...]
        o_ref[...] = 1.0 / jnp.sqrt(jnp.maximum(d[:, :1], 1.0))

    return pl.pallas_call(
        body,
        grid=(nb,),
        in_specs=[pl.BlockSpec((bn, R), lambda i: (i, 0)),
                  pl.BlockSpec((bn, R), lambda i: (i + nb, 0))],
        out_specs=pl.BlockSpec((bn, 1), lambda i: (i, 0)),
        out_shape=jax.ShapeDtypeStruct((n, 1), _f32),
    )(degS, degS)


def _tc_scale(x, dinv, split=False):
    rows, R = x.shape
    n = dinv.shape[0]
    bn = min(n, 512)
    nb = n // bn
    Rh = R // 2

    def body(x_ref, d_ref, *o_refs):
        u = x_ref[...] * d_ref[...]
        if split:
            o_refs[0][...] = u[:, :Rh]
            o_refs[1][...] = u[:, Rh:]
        else:
            o_refs[0][...] = u

    if split:
        out_specs = [pl.BlockSpec((bn, Rh), lambda i: (i, 0))] * 2
        out_shape = [jax.ShapeDtypeStruct((rows, Rh), _f32)] * 2
    else:
        out_specs = [pl.BlockSpec((bn, R), lambda i: (i, 0))]
        out_shape = [jax.ShapeDtypeStruct((rows, R), _f32)]
    res = pl.pallas_call(
        body,
        grid=(rows // bn,),
        in_specs=[pl.BlockSpec((bn, R), lambda i: (i, 0)),
                  pl.BlockSpec((bn, 1), lambda i: (i % nb, 0))],
        out_specs=out_specs,
        out_shape=out_shape,
    )(x, dinv)
    return res if split else res[0]


def _tc_step2(SA, SB, Txm2, dinv, coef, want_u):
    n = dinv.shape[0]
    Rh = SA.shape[1]
    bn = min(n, 512)
    nb = n // bn

    def body(*refs):
        sa1, sa2, sb1, sb2 = refs[0], refs[1], refs[2], refs[3]
        i = 4
        t_ref = refs[i] if Txm2 is not None else None
        if Txm2 is not None:
            i += 1
        d_ref = refs[i]; i += 1
        outs = refs[i:]
        d = d_ref[...]
        txl = coef * d * (sa1[...] + sa2[...])
        txr = coef * d * (sb1[...] + sb2[...])
        if t_ref is not None:
            t = t_ref[...]
            txl = txl - t[:, :Rh]
            txr = txr - t[:, Rh:]
        outs[0][...] = jnp.concatenate([txl, txr], axis=1)
        if want_u:
            outs[1][...] = d * txl
            outs[2][...] = d * txr

    in_specs = [pl.BlockSpec((bn, Rh), lambda i: (i, 0)),
                pl.BlockSpec((bn, Rh), lambda i: (i + nb, 0)),
                pl.BlockSpec((bn, Rh), lambda i: (i, 0)),
                pl.BlockSpec((bn, Rh), lambda i: (i + nb, 0))]
    args = [SA, SA, SB, SB]
    if Txm2 is not None:
        in_specs.append(pl.BlockSpec((bn, 2 * Rh), lambda i: (i, 0)))
        args.append(Txm2)
    in_specs.append(pl.BlockSpec((bn, 1), lambda i: (i, 0)))
    args.append(dinv)

    out_specs = [pl.BlockSpec((bn, 2 * Rh), lambda i: (i, 0))]
    out_shape = [jax.ShapeDtypeStruct((n, 2 * Rh), _f32)]
    if want_u:
        out_specs += [pl.BlockSpec((bn, Rh), lambda i: (i, 0))] * 2
        out_shape += [jax.ShapeDtypeStruct((n, Rh), _f32)] * 2

    res = pl.pallas_call(
        body,
        grid=(nb,),
        in_specs=in_specs,
        out_specs=out_specs,
        out_shape=out_shape,
    )(*args)
    return res if want_u else (res[0], None, None)


def _tc_step(S, Txm2, dinv, coef, want_u, two_slabs, V=None):
    n = dinv.shape[0]
    R = S.shape[1]
    rows = S.shape[0] // 2 if two_slabs else S.shape[0]
    bn = min(n, 512)
    nb = n // bn
    nblocks = rows // bn

    def body(*refs):
        i = 0
        sa_ref = refs[i]; i += 1
        sb_ref = refs[i] if two_slabs else None
        if two_slabs:
            i += 1
        t_ref = refs[i] if Txm2 is not None else None
        if Txm2 is not None:
            i += 1
        v_ref = refs[i] if V is not None else None
        if V is not None:
            i += 1
        d_ref = refs[i]; i += 1
        outs = refs[i:]
        stot = sa_ref[...] + sb_ref[...] if two_slabs else sa_ref[...]
        tx = coef * d_ref[...] * stot
        if t_ref is not None:
            tx = tx - t_ref[...]
        if v_ref is not None:
            tx = tx + v_ref[...]
        outs[0][...] = tx
        if want_u:
            outs[1][...] = d_ref[...] * tx

    in_specs = [pl.BlockSpec((bn, R), lambda i: (i, 0))]
    args = [S]
    if two_slabs:
        in_specs.append(pl.BlockSpec((bn, R), lambda i: (i + nblocks, 0)))
        args.append(S)
    if Txm2 is not None:
        in_specs.append(pl.BlockSpec((bn, R), lambda i: (i, 0)))
        args.append(Txm2)
    if V is not None:
        in_specs.append(pl.BlockSpec((bn, R), lambda i: (i, 0)))
        args.append(V)
    in_specs.append(pl.BlockSpec((bn, 1), lambda i: (i % nb, 0)))
    args.append(dinv)

    out_shape = [jax.ShapeDtypeStruct((rows, R), _f32)]
    out_specs = [pl.BlockSpec((bn, R), lambda i: (i, 0))]
    if want_u:
        out_shape.append(jax.ShapeDtypeStruct((rows, R), _f32))
        out_specs.append(pl.BlockSpec((bn, R), lambda i: (i, 0)))

    res = pl.pallas_call(
        body,
        grid=(nblocks,),
        in_specs=in_specs,
        out_specs=out_specs,
        out_shape=out_shape,
    )(*args)
    return res if want_u else (res[0], None)


def _tc_combine(txs, Wbig, bvec, relu, W2big=None, emit_main=True):
    M, G = txs[0].shape
    Gout = Wbig.shape[2]
    bm = min(M, 512)

    def body(*refs):
        t_refs = refs[:6]
        w_ref, b_ref = refs[6], refs[7]
        i = 8
        w2_ref = refs[i] if W2big is not None else None
        if W2big is not None:
            i += 1
        outs = refs[i:]
        acc = jnp.dot(t_refs[0][...], w_ref[0], preferred_element_type=_f32)
        for k in range(1, 6):
            acc = acc + jnp.dot(t_refs[k][...], w_ref[k],
                                preferred_element_type=_f32)
        acc = acc + b_ref[...]
        if relu:
            acc = jnp.maximum(acc, 0.0)
        j = 0
        if emit_main:
            outs[j][...] = acc
            j += 1
        if W2big is not None:
            outs[j][...] = jnp.dot(acc, w2_ref[...],
                                   preferred_element_type=_f32)

    in_specs = [pl.BlockSpec((bm, G), lambda i: (i, 0)) for _ in range(6)]
    in_specs.append(pl.BlockSpec((6, G, Gout), lambda i: (0, 0, 0)))
    in_specs.append(pl.BlockSpec((1, Gout), lambda i: (0, 0)))
    args = list(txs) + [Wbig, bvec]
    out_specs, out_shape = [], []
    if emit_main:
        out_specs.append(pl.BlockSpec((bm, Gout), lambda i: (i, 0)))
        out_shape.append(jax.ShapeDtypeStruct((M, Gout), _f32))
    if W2big is not None:
        g2 = W2big.shape[1]
        in_specs.append(pl.BlockSpec((Gout, g2), lambda i: (0, 0)))
        args.append(W2big)
        out_specs.append(pl.BlockSpec((bm, g2), lambda i: (i, 0)))
        out_shape.append(jax.ShapeDtypeStruct((M, g2), _f32))

    res = pl.pallas_call(
        body,
        grid=(M // bm,),
        in_specs=in_specs,
        out_specs=out_specs,
        out_shape=out_shape,
    )(*args)
    return res[0] if len(res) == 1 else res


def _tc_latent(h, We3, be, Wd3, bd3):
    def body(h_ref, we_ref, be_ref, wd_ref, bd_ref, o_ref):
        acc = jnp.broadcast_to(be_ref[...], (_B, _Z))
        for nn in range(_Z):
            acc = acc + jnp.dot(h_ref[nn], we_ref[nn],
                                preferred_element_type=_f32)
        z = jnp.maximum(acc, 0.0)
        for nn in range(_Z):
            o_ref[nn] = jnp.maximum(
                jnp.dot(z, wd_ref[nn], preferred_element_type=_f32)
                + bd_ref[nn], 0.0)

    return pl.pallas_call(
        body,
        out_shape=jax.ShapeDtypeStruct((_Z, _B, 32), _f32),
    )(h, We3, be.reshape(1, _Z), Wd3, bd3)


def _cheb_layer(h_flat, edge, dinv1, W, b, relu, n, C, W2=None):
    E = edge.shape[1]
    R = h_flat.shape[1]
    f_in = W.shape[1]
    two = C == 1
    split = C == 1 and n * R >= 1 << 20

    if split:
        Rh = R // 2
        ua, ub = _tc_scale(h_flat, dinv1, split=True)
        SA = _spmv(ua, edge, n, Rh, E, 1)
        SB = _spmv(ub, edge, n, Rh, E, 1)
        Tx1, ua, ub = _tc_step2(SA, SB, None, dinv1, -1.0, True)
        txs = [h_flat, Tx1]
        for k in range(2, _K):
            SA = _spmv(ua, edge, n, Rh, E, 1)
            SB = _spmv(ub, edge, n, Rh, E, 1)
            Txk, ua, ub = _tc_step2(SA, SB, txs[k - 2], dinv1, -2.0,
                                    k < _K - 1)
            txs.append(Txk)
    else:
        u0 = _tc_scale(h_flat, dinv1)
        S0 = _spmv(u0, edge, n, R, E, C)
        Tx1, u = _tc_step(S0, None, dinv1, -1.0, True, two)
        txs = [h_flat, Tx1]
        for k in range(2, _K):
            Sk = _spmv(u, edge, n, R, E, C)
            Txk, u = _tc_step(Sk, txs[k - 2], dinv1, -2.0, k < _K - 1, two)
            txs.append(Txk)

    g = W.shape[2]
    bc = R // f_in
    eye = jnp.eye(bc, dtype=_f32)
    Wbig = (eye[None, :, None, :, None] *
            W[:, None, :, None, :]).reshape(_K, bc * f_in, bc * g)
    bb = jnp.zeros((g,), _f32) if b is None else b
    bvec = jnp.tile(bb, bc).reshape(1, bc * g)
    return _tc_combine(txs, Wbig, bvec, relu, W2big=W2,
                       emit_main=(W2 is None))


def _tc_clenstep(S, bm2, dinv, coef, want_u, Vs, k, C):
    n = dinv.shape[0]
    w = Vs[0].shape[1] // 6
    R = w * C * len(Vs)
    bn = min(n, 512)
    nb_ = n // bn

    def body(*refs):
        i = 0
        if S is not None:
            sa, sb = refs[0], refs[1]
            i = 2
        t_ref = refs[i] if bm2 is not None else None
        if bm2 is not None:
            i += 1
        nv = C * len(Vs)
        v_refs = refs[i:i + nv]; i += nv
        d_ref = refs[i]; i += 1
        outs = refs[i:]
        v = jnp.concatenate(
            [vr[:, w * k:w * (k + 1)] for vr in v_refs], axis=1)
        d = d_ref[...]
        if S is not None:
            b_ = coef * d * (sa[...] + sb[...]) + v
        else:
            b_ = v
        if t_ref is not None:
            b_ = b_ - t_ref[...]
        outs[0][...] = b_
        if want_u:
            outs[1][...] = d * b_

    in_specs, args = [], []
    if S is not None:
        in_specs += [pl.BlockSpec((bn, R), lambda i: (i, 0)),
                     pl.BlockSpec((bn, R), lambda i: (i + nb_, 0))]
        args += [S, S]
    if bm2 is not None:
        in_specs.append(pl.BlockSpec((bn, R), lambda i: (i, 0)))
        args.append(bm2)
    for V in Vs:
        for c in range(C):
            in_specs.append(
                pl.BlockSpec((bn, 6 * w), lambda i, c=c: (i + c * nb_, 0)))
            args.append(V)
    in_specs.append(pl.BlockSpec((bn, 1), lambda i: (i, 0)))
    args.append(dinv)
    out_specs = [pl.BlockSpec((bn, R), lambda i: (i, 0))]
    out_shape = [jax.ShapeDtypeStruct((n, R), _f32)]
    if want_u:
        out_specs.append(pl.BlockSpec((bn, R), lambda i: (i, 0)))
        out_shape.append(jax.ShapeDtypeStruct((n, R), _f32))
    res = pl.pallas_call(
        body,
        grid=(n // bn,),
        in_specs=in_specs,
        out_specs=out_specs,
        out_shape=out_shape,
    )(*args)
    return res if want_u else (res[0], None)


def _cheb_clenshaw(Vs, edge, dinv1, n, C):
    E = edge.shape[1]
    R = (Vs[0].shape[1] // 6) * C * len(Vs)
    b_k1, u = _tc_clenstep(None, None, dinv1, 0.0, True, Vs, 5, C)
    b_k2 = None
    for k in range(4, 0, -1):
        S = _spmv(u, edge, n, R, E, 1)
        b_k, u = _tc_clenstep(S, b_k2, dinv1, -2.0, True, Vs, k, C)
        b_k2, b_k1 = b_k1, b_k
    S = _spmv(u, edge, n, R, E, 1)
    out, _ = _tc_clenstep(S, b_k2, dinv1, -1.0, False, Vs, 0, C)
    return out


def _level_dinv(edge, n):
    E = edge.shape[1]
    R = 48 if n == _N[0] else 16
    ones = jnp.ones((n, R), _f32)
    degS = _spmv(ones, edge, n, R, E)
    return _tc_dinv(degS, n)


def kernel(x, edges, down_idx, up_idx, enc_W, enc_b, dec_W, dec_b,
           lin_enc_W, lin_enc_b, lin_dec_W, lin_dec_b):
    n0 = _N[0]
    dinvs = [None] * 4

    def dinv_for(lvl):
        if dinvs[lvl] is None:
            dinvs[lvl] = _level_dinv(edges[lvl], _N[lvl])
        return dinvs[lvl]

    h = x.reshape(_B, n0, _ENC_F[0]).transpose(1, 0, 2).reshape(
        n0, _B * _ENC_F[0])
    for i in range(4):
        out = _cheb_layer(h, edges[i], dinv_for(i), enc_W[i], enc_b[i],
                          True, _N[i], 1)
        h = _gather_rows(out, down_idx[i], _N[i + 1])

    We3 = lin_enc_W.reshape(_Z, 32, _Z)
    Wd3 = lin_dec_W.reshape(_Z, _Z, 32).swapaxes(0, 1)
    bd3 = lin_dec_b.reshape(_Z, 1, 32)
    h = _tc_latent(h.reshape(_Z, _B, 32), We3, lin_enc_b, Wd3, bd3)
    h = h.reshape(_Z, _B * 32)

    for i in range(4):
        lvl = 3 - i
        n = _N[lvl]
        f_in, f_out = _DEC[i]
        if lvl > 0:
            hu = _gather_rows(h, up_idx[lvl], n)
            h = _cheb_layer(hu, edges[lvl], dinv_for(lvl), dec_W[i],
                            dec_b[i], True, n, 1)
        else:
            C, bc = 4, 4
            hu = _gather_rows(h.reshape(_N[1] * C, bc * f_in),
                              up_idx[0], n, C=C, scale=C)
            Wf = dec_W[4]
            W2big = (jnp.eye(bc, dtype=_f32)[:, None, None, :, None] *
                     Wf.transpose(1, 0, 2)[None, :, :, None, :]).reshape(
                         bc * 16, _K * bc * _ENC_F[0])
            V = _cheb_layer(hu, edges[0], dinv_for(0), dec_W[i],
                            dec_b[i], True, n, C, W2=W2big)
            res = _cheb_clenshaw([V], edges[0], dinv_for(0), n, C)
            return res.reshape(n, _B, _ENC_F[0]).transpose(1, 0, 2).reshape(
                _B * n, _ENC_F[0])
